# baseline (speedup 1.0000x reference)
import dataclasses
import functools

import jax
import jax.numpy as jnp
from jax import lax
from jax.experimental import pallas as pl
from jax.experimental.pallas import tpu as pltpu
from jax.experimental.pallas import tpu_sc as plsc

_VOCAB = 100000
_DIM = 768
_MAXLEN = 128
_BATCH = 1024
_SEQ = 128
_N = _BATCH * _SEQ

_NC = 2
_NS = 16
_NW = _NC * _NS
_B_PER_W = _N // _NW
_CHUNK = 32
_NBUF = 2
_DIM2 = _DIM // 2


def _sc_gather(table, idx, n_rows):
    b_per_w = n_rows // _NW
    n_chunks = b_per_w // _CHUNK
    mesh = plsc.VectorSubcoreMesh(
        core_axis_name="c", subcore_axis_name="s", num_cores=_NC, num_subcores=_NS
    )

    @functools.partial(
        pl.kernel,
        out_type=jax.ShapeDtypeStruct((n_rows, _DIM2), jnp.uint32),
        mesh=mesh,
        compiler_params=dataclasses.replace(
            pltpu.CompilerParams(), needs_layout_passes=False
        ),
        scratch_types=[
            pltpu.VMEM((b_per_w,), jnp.int32),
        ]
        + [pltpu.VMEM((_CHUNK, _DIM), jnp.float32) for _ in range(_NBUF)]
        + [pltpu.VMEM((_CHUNK, _DIM2), jnp.uint32) for _ in range(_NBUF)]
        + [pltpu.SemaphoreType.DMA for _ in range(2 * _NBUF)],
    )
    def k(table_hbm, idx_hbm, out_hbm, idx_v, *bufs_and_sems):
        rows = bufs_and_sems[:_NBUF]
        words = bufs_and_sems[_NBUF : 2 * _NBUF]
        gsems = bufs_and_sems[2 * _NBUF : 3 * _NBUF]
        wsems = bufs_and_sems[3 * _NBUF :]
        wid = lax.axis_index("s") * _NC + lax.axis_index("c")
        base = wid * b_per_w
        pltpu.sync_copy(idx_hbm.at[pl.ds(base, b_per_w)], idx_v)
        rnd = jnp.uint32(0x8000)
        msk = jnp.uint32(0xFFFF0000)

        def start_gather(c, buf, gsem):
            pltpu.make_async_copy(
                table_hbm.at[idx_v.at[pl.ds(c * _CHUNK, _CHUNK)]], buf, gsem
            ).start()

        for b in range(_NBUF):
            start_gather(b, rows[b], gsems[b])

        @pl.loop(0, n_chunks, step=_NBUF)
        def _(c):
            for b in range(_NBUF):
                cur = c + b
                fbuf, wbuf, gsem, wsem = rows[b], words[b], gsems[b], wsems[b]
                pltpu.make_async_copy(
                    table_hbm.at[idx_v.at[pl.ds(0, _CHUNK)]], fbuf, gsem
                ).wait()

                @pl.loop(0, _CHUNK)
                def _(r):
                    for j in range(_DIM2 // 16):
                        ev = fbuf[r, pl.ds(16 * j, 16)]
                        ov = fbuf[r, pl.ds(16 * j + _DIM2, 16)]
                        eu = plsc.bitcast(ev, jnp.uint32)
                        ou = plsc.bitcast(ov, jnp.uint32)
                        w = ((eu + rnd) >> jnp.uint32(16)) | ((ou + rnd) & msk)
                        wbuf[r, pl.ds(16 * j, 16)] = w

                pltpu.make_async_copy(
                    wbuf, out_hbm.at[pl.ds(base + cur * _CHUNK, _CHUNK)], wsem
                ).start()

                @pl.when(cur + _NBUF < n_chunks)
                def _():
                    pltpu.make_async_copy(
                        wbuf, out_hbm.at[pl.ds(base, _CHUNK)], wsem
                    ).wait()
                    start_gather(cur + _NBUF, fbuf, gsem)

        for b in range(_NBUF):
            pltpu.make_async_copy(
                words[b], out_hbm.at[pl.ds(base, _CHUNK)], wsems[b]
            ).wait()

    return k(table, idx)


_BR = 512


def _ln_body(seg_ref, emb_ref, pos_ref, segemb_ref, gamma_ref, beta_ref, out_ref):
    w = emb_ref[...]
    he = lax.bitcast_convert_type(w << jnp.uint32(16), jnp.float32)
    ho = lax.bitcast_convert_type(w & jnp.uint32(0xFFFF0000), jnp.float32)
    h = jnp.concatenate([he, ho], axis=1)
    pos = pos_ref[...]
    posb = jnp.broadcast_to(pos[None], (_BR // _SEQ, _SEQ, _DIM)).reshape(_BR, _DIM)
    s = seg_ref[0, 0, :].astype(jnp.float32)
    se = segemb_ref[...]
    sadd = se[0][None, :] + s[:, None] * (se[1] - se[0])[None, :]
    h = h + posb + sadd
    mean = jnp.mean(h, axis=1, keepdims=True)
    hc = h - mean
    var = jnp.mean(hc * hc, axis=1, keepdims=True)
    out_ref[...] = hc * lax.rsqrt(var + 1e-5) * gamma_ref[...] + beta_ref[...]


def _ln_body_buf(buf_ref, seg_ref, emb_ref, pos_ref, segemb_ref, gamma_ref,
                 beta_ref, out_ref):
    del buf_ref
    _ln_body(seg_ref, emb_ref, pos_ref, segemb_ref, gamma_ref, beta_ref, out_ref)


_NCHUNKS_OUTER = 8
_NCH = _N // _NCHUNKS_OUTER


def _tc_ln_chunk(buf, emb_c, seg3, c, pos_embed, seg_embed, gamma2, beta2):
    nblk_c = _NCH // _BR
    blk0 = c * nblk_c
    common_in_specs = [
        pl.BlockSpec((1, 1, _BR), lambda i, b=blk0: (b + i, 0, 0)),
        pl.BlockSpec((_BR, _DIM2), lambda i: (i, 0)),
        pl.BlockSpec((_SEQ, _DIM), lambda i: (0, 0)),
        pl.BlockSpec((2, _DIM), lambda i: (0, 0)),
        pl.BlockSpec((1, _DIM), lambda i: (0, 0)),
        pl.BlockSpec((1, _DIM), lambda i: (0, 0)),
    ]
    out_spec = pl.BlockSpec((_BR, _DIM), lambda i, b=blk0: (b + i, 0))
    out_shape = jax.ShapeDtypeStruct((_N, _DIM), jnp.float32)
    args = (seg3, emb_c, pos_embed, seg_embed, gamma2, beta2)
    cp = pltpu.CompilerParams(dimension_semantics=("parallel",))
    if buf is None:
        return pl.pallas_call(
            _ln_body,
            grid=(nblk_c,),
            in_specs=common_in_specs,
            out_specs=out_spec,
            out_shape=out_shape,
            compiler_params=cp,
        )(*args)
    return pl.pallas_call(
        _ln_body_buf,
        grid=(nblk_c,),
        in_specs=[pl.BlockSpec(memory_space=pl.ANY)] + common_in_specs,
        out_specs=out_spec,
        out_shape=out_shape,
        input_output_aliases={0: 0},
        compiler_params=cp,
    )(buf, *args)


@jax.jit
def kernel(x, seg, tok_embed, pos_embed, seg_embed, ln_gamma, ln_beta):
    xf = x.reshape(_N)
    seg3 = seg.reshape(_N // _BR, 1, _BR)
    gamma2 = ln_gamma.reshape(1, _DIM)
    beta2 = ln_beta.reshape(1, _DIM)
    embs = [
        _sc_gather(tok_embed, lax.slice(xf, (c * _NCH,), ((c + 1) * _NCH,)), _NCH)
        for c in range(_NCHUNKS_OUTER)
    ]
    buf = None
    for c in range(_NCHUNKS_OUTER):
        buf = _tc_ln_chunk(buf, embs[c], seg3, c, pos_embed, seg_embed,
                           gamma2, beta2)
    return buf.reshape(_BATCH, _SEQ, _DIM)


# speedup vs baseline: 1.2819x; 1.2819x over previous
import functools

import jax
import jax.numpy as jnp
from jax import lax
from jax.experimental import pallas as pl
from jax.experimental.pallas import tpu as pltpu
from jax.experimental.pallas import tpu_sc as plsc

_VOCAB = 100000
_DIM = 768
_MAXLEN = 128
_BATCH = 1024
_SEQ = 128
_N = _BATCH * _SEQ

_NC = 2
_NS = 16
_NW = _NC * _NS
_B_PER_W = _N // _NW
_CHUNK = 32
_NBUF = 4


def _sc_gather(table, idx, n_rows):
    b_per_w = n_rows // _NW
    n_chunks = b_per_w // _CHUNK
    mesh = plsc.VectorSubcoreMesh(
        core_axis_name="c", subcore_axis_name="s", num_cores=_NC, num_subcores=_NS
    )

    @functools.partial(
        pl.kernel,
        out_type=jax.ShapeDtypeStruct((n_rows, _DIM), jnp.float32),
        mesh=mesh,
        scratch_types=[
            pltpu.VMEM((b_per_w,), jnp.int32),
        ]
        + [pltpu.VMEM((_CHUNK, _DIM), jnp.float32) for _ in range(_NBUF)]
        + [pltpu.SemaphoreType.DMA for _ in range(2 * _NBUF)],
    )
    def k(table_hbm, idx_hbm, out_hbm, idx_v, *bufs_and_sems):
        rows = bufs_and_sems[:_NBUF]
        gsems = bufs_and_sems[_NBUF : 2 * _NBUF]
        wsems = bufs_and_sems[2 * _NBUF :]
        wid = lax.axis_index("s") * _NC + lax.axis_index("c")
        base = wid * b_per_w
        pltpu.sync_copy(idx_hbm.at[pl.ds(base, b_per_w)], idx_v)

        def start_gather(c, buf, gsem):
            pltpu.make_async_copy(
                table_hbm.at[idx_v.at[pl.ds(c * _CHUNK, _CHUNK)]], buf, gsem
            ).start()

        for b in range(_NBUF):
            start_gather(b, rows[b], gsems[b])

        @pl.loop(0, n_chunks, step=_NBUF)
        def _(c):
            for b in range(_NBUF):
                cur = c + b
                buf, gsem, wsem = rows[b], gsems[b], wsems[b]
                pltpu.make_async_copy(
                    table_hbm.at[idx_v.at[pl.ds(0, _CHUNK)]], buf, gsem
                ).wait()
                pltpu.make_async_copy(
                    buf, out_hbm.at[pl.ds(base + cur * _CHUNK, _CHUNK)], wsem
                ).start()

                @pl.when(cur + _NBUF < n_chunks)
                def _():
                    pltpu.make_async_copy(
                        buf, out_hbm.at[pl.ds(base, _CHUNK)], wsem
                    ).wait()
                    start_gather(cur + _NBUF, buf, gsem)

        for b in range(_NBUF):
            pltpu.make_async_copy(
                rows[b], out_hbm.at[pl.ds(base, _CHUNK)], wsems[b]
            ).wait()

    return k(table, idx)


_BR = 512


def _ln_body(seg_ref, emb_ref, pos_ref, segemb_ref, gamma_ref, beta_ref, out_ref):
    h = emb_ref[...]
    pos = pos_ref[...]
    posb = jnp.broadcast_to(pos[None], (_BR // _SEQ, _SEQ, _DIM)).reshape(_BR, _DIM)
    s = seg_ref[0, 0, :].astype(jnp.float32)
    se = segemb_ref[...]
    sadd = se[0][None, :] + s[:, None] * (se[1] - se[0])[None, :]
    h = h + posb + sadd
    mean = jnp.mean(h, axis=1, keepdims=True)
    hc = h - mean
    var = jnp.mean(hc * hc, axis=1, keepdims=True)
    out_ref[...] = hc * lax.rsqrt(var + 1e-5) * gamma_ref[...] + beta_ref[...]


def _ln_body_buf(buf_ref, seg_ref, emb_ref, pos_ref, segemb_ref, gamma_ref,
                 beta_ref, out_ref):
    del buf_ref
    _ln_body(seg_ref, emb_ref, pos_ref, segemb_ref, gamma_ref, beta_ref, out_ref)


_NCHUNKS_OUTER = 16
_NCH = _N // _NCHUNKS_OUTER


def _tc_ln_chunk(buf, emb_c, seg3, c, pos_embed, seg_embed, gamma2, beta2):
    nblk_c = _NCH // _BR
    blk0 = c * nblk_c
    common_in_specs = [
        pl.BlockSpec((1, 1, _BR), lambda i, b=blk0: (b + i, 0, 0)),
        pl.BlockSpec((_BR, _DIM), lambda i: (i, 0)),
        pl.BlockSpec((_SEQ, _DIM), lambda i: (0, 0)),
        pl.BlockSpec((2, _DIM), lambda i: (0, 0)),
        pl.BlockSpec((1, _DIM), lambda i: (0, 0)),
        pl.BlockSpec((1, _DIM), lambda i: (0, 0)),
    ]
    out_spec = pl.BlockSpec((_BR, _DIM), lambda i, b=blk0: (b + i, 0))
    out_shape = jax.ShapeDtypeStruct((_N, _DIM), jnp.float32)
    args = (seg3, emb_c, pos_embed, seg_embed, gamma2, beta2)
    cp = pltpu.CompilerParams(dimension_semantics=("parallel",))
    if buf is None:
        return pl.pallas_call(
            _ln_body,
            grid=(nblk_c,),
            in_specs=common_in_specs,
            out_specs=out_spec,
            out_shape=out_shape,
            compiler_params=cp,
        )(*args)
    return pl.pallas_call(
        _ln_body_buf,
        grid=(nblk_c,),
        in_specs=[pl.BlockSpec(memory_space=pl.ANY)] + common_in_specs,
        out_specs=out_spec,
        out_shape=out_shape,
        input_output_aliases={0: 0},
        compiler_params=cp,
    )(buf, *args)


@jax.jit
def kernel(x, seg, tok_embed, pos_embed, seg_embed, ln_gamma, ln_beta):
    xf = x.reshape(_N)
    seg3 = seg.reshape(_N // _BR, 1, _BR)
    gamma2 = ln_gamma.reshape(1, _DIM)
    beta2 = ln_beta.reshape(1, _DIM)
    embs = [
        _sc_gather(tok_embed, lax.slice(xf, (c * _NCH,), ((c + 1) * _NCH,)), _NCH)
        for c in range(_NCHUNKS_OUTER)
    ]
    buf = None
    for c in range(_NCHUNKS_OUTER):
        buf = _tc_ln_chunk(buf, embs[c], seg3, c, pos_embed, seg_embed,
                           gamma2, beta2)
    return buf.reshape(_BATCH, _SEQ, _DIM)


# speedup vs baseline: 1.2878x; 1.0046x over previous
import functools

import jax
import jax.numpy as jnp
from jax import lax
from jax.experimental import pallas as pl
from jax.experimental.pallas import tpu as pltpu
from jax.experimental.pallas import tpu_sc as plsc

_VOCAB = 100000
_DIM = 768
_MAXLEN = 128
_BATCH = 1024
_SEQ = 128
_N = _BATCH * _SEQ

_NC = 2
_NS = 16
_NW = _NC * _NS
_B_PER_W = _N // _NW
_CHUNK = 32
_NBUF = 4


def _sc_gather(table, idx, n_rows):
    b_per_w = n_rows // _NW
    n_chunks = b_per_w // _CHUNK
    mesh = plsc.VectorSubcoreMesh(
        core_axis_name="c", subcore_axis_name="s", num_cores=_NC, num_subcores=_NS
    )

    @functools.partial(
        pl.kernel,
        out_type=jax.ShapeDtypeStruct((n_rows, _DIM), jnp.float32),
        mesh=mesh,
        scratch_types=[
            pltpu.VMEM((b_per_w,), jnp.int32),
        ]
        + [pltpu.VMEM((_CHUNK, _DIM), jnp.float32) for _ in range(_NBUF)]
        + [pltpu.SemaphoreType.DMA for _ in range(2 * _NBUF)],
    )
    def k(table_hbm, idx_hbm, out_hbm, idx_v, *bufs_and_sems):
        rows = bufs_and_sems[:_NBUF]
        gsems = bufs_and_sems[_NBUF : 2 * _NBUF]
        wsems = bufs_and_sems[2 * _NBUF :]
        wid = lax.axis_index("s") * _NC + lax.axis_index("c")
        base = wid * b_per_w
        pltpu.sync_copy(idx_hbm.at[pl.ds(base, b_per_w)], idx_v)

        def start_gather(c, buf, gsem):
            pltpu.make_async_copy(
                table_hbm.at[idx_v.at[pl.ds(c * _CHUNK, _CHUNK)]], buf, gsem
            ).start()

        for b in range(_NBUF):
            start_gather(b, rows[b], gsems[b])

        @pl.loop(0, n_chunks, step=_NBUF)
        def _(c):
            for b in range(_NBUF):
                cur = c + b
                buf, gsem, wsem = rows[b], gsems[b], wsems[b]
                pltpu.make_async_copy(
                    table_hbm.at[idx_v.at[pl.ds(0, _CHUNK)]], buf, gsem
                ).wait()
                pltpu.make_async_copy(
                    buf, out_hbm.at[pl.ds(base + cur * _CHUNK, _CHUNK)], wsem
                ).start()

                @pl.when(cur + _NBUF < n_chunks)
                def _():
                    pltpu.make_async_copy(
                        buf, out_hbm.at[pl.ds(base, _CHUNK)], wsem
                    ).wait()
                    start_gather(cur + _NBUF, buf, gsem)

        for b in range(_NBUF):
            pltpu.make_async_copy(
                rows[b], out_hbm.at[pl.ds(base, _CHUNK)], wsems[b]
            ).wait()

    return k(table, idx)


_BR = 512


def _ln_body(seg_ref, emb_ref, pos_ref, segemb_ref, gamma_ref, beta_ref, out_ref):
    h = emb_ref[...]
    pos = pos_ref[...]
    posb = jnp.broadcast_to(pos[None], (_BR // _SEQ, _SEQ, _DIM)).reshape(_BR, _DIM)
    s = seg_ref[0, 0, :].astype(jnp.float32)
    se = segemb_ref[...]
    sadd = se[0][None, :] + s[:, None] * (se[1] - se[0])[None, :]
    h = h + posb + sadd
    mean = jnp.mean(h, axis=1, keepdims=True)
    hc = h - mean
    var = jnp.mean(hc * hc, axis=1, keepdims=True)
    out_ref[...] = hc * lax.rsqrt(var + 1e-5) * gamma_ref[...] + beta_ref[...]


def _ln_body_buf(buf_ref, seg_ref, emb_ref, pos_ref, segemb_ref, gamma_ref,
                 beta_ref, out_ref):
    del buf_ref
    _ln_body(seg_ref, emb_ref, pos_ref, segemb_ref, gamma_ref, beta_ref, out_ref)


_SIZES = [8192] + [16384] * 7 + [8192]


def _tc_ln_chunk(buf, emb_c, seg3, row_off, size, pos_embed, seg_embed,
                 gamma2, beta2):
    nblk_c = size // _BR
    blk0 = row_off // _BR
    common_in_specs = [
        pl.BlockSpec((1, 1, _BR), lambda i, b=blk0: (b + i, 0, 0)),
        pl.BlockSpec((_BR, _DIM), lambda i: (i, 0)),
        pl.BlockSpec((_SEQ, _DIM), lambda i: (0, 0)),
        pl.BlockSpec((2, _DIM), lambda i: (0, 0)),
        pl.BlockSpec((1, _DIM), lambda i: (0, 0)),
        pl.BlockSpec((1, _DIM), lambda i: (0, 0)),
    ]
    out_spec = pl.BlockSpec((_BR, _DIM), lambda i, b=blk0: (b + i, 0))
    out_shape = jax.ShapeDtypeStruct((_N, _DIM), jnp.float32)
    args = (seg3, emb_c, pos_embed, seg_embed, gamma2, beta2)
    cp = pltpu.CompilerParams(dimension_semantics=("parallel",))
    if buf is None:
        return pl.pallas_call(
            _ln_body,
            grid=(nblk_c,),
            in_specs=common_in_specs,
            out_specs=out_spec,
            out_shape=out_shape,
            compiler_params=cp,
        )(*args)
    return pl.pallas_call(
        _ln_body_buf,
        grid=(nblk_c,),
        in_specs=[pl.BlockSpec(memory_space=pl.ANY)] + common_in_specs,
        out_specs=out_spec,
        out_shape=out_shape,
        input_output_aliases={0: 0},
        compiler_params=cp,
    )(buf, *args)


@jax.jit
def kernel(x, seg, tok_embed, pos_embed, seg_embed, ln_gamma, ln_beta):
    xf = x.reshape(_N)
    seg3 = seg.reshape(_N // _BR, 1, _BR)
    gamma2 = ln_gamma.reshape(1, _DIM)
    beta2 = ln_beta.reshape(1, _DIM)
    offs = [0]
    for s in _SIZES:
        offs.append(offs[-1] + s)
    embs = [
        _sc_gather(tok_embed, lax.slice(xf, (off,), (off + size,)), size)
        for off, size in zip(offs, _SIZES)
    ]
    buf = None
    for emb_c, off, size in zip(embs, offs, _SIZES):
        buf = _tc_ln_chunk(buf, emb_c, seg3, off, size, pos_embed, seg_embed,
                           gamma2, beta2)
    return buf.reshape(_BATCH, _SEQ, _DIM)


# speedup vs baseline: 1.3523x; 1.0501x over previous
import functools

import jax
import jax.numpy as jnp
from jax import lax
from jax.experimental import pallas as pl
from jax.experimental.pallas import tpu as pltpu
from jax.experimental.pallas import tpu_sc as plsc

_VOCAB = 100000
_DIM = 768
_MAXLEN = 128
_BATCH = 1024
_SEQ = 128
_N = _BATCH * _SEQ

_NC = 2
_NS = 16
_NW = _NC * _NS
_B_PER_W = _N // _NW
_CHUNK = 32
_NBUF = 4


def _sc_gather(table, idx, n_rows):
    b_per_w = n_rows // _NW
    n_chunks = b_per_w // _CHUNK
    mesh = plsc.VectorSubcoreMesh(
        core_axis_name="c", subcore_axis_name="s", num_cores=_NC, num_subcores=_NS
    )

    @functools.partial(
        pl.kernel,
        out_type=jax.ShapeDtypeStruct((n_rows, _DIM), jnp.float32),
        mesh=mesh,
        scratch_types=[
            pltpu.VMEM((b_per_w,), jnp.int32),
        ]
        + [pltpu.VMEM((_CHUNK, _DIM), jnp.float32) for _ in range(_NBUF)]
        + [pltpu.SemaphoreType.DMA for _ in range(2 * _NBUF)],
    )
    def k(table_hbm, idx_hbm, out_hbm, idx_v, *bufs_and_sems):
        rows = bufs_and_sems[:_NBUF]
        gsems = bufs_and_sems[_NBUF : 2 * _NBUF]
        wsems = bufs_and_sems[2 * _NBUF :]
        wid = lax.axis_index("s") * _NC + lax.axis_index("c")
        base = wid * b_per_w
        pltpu.sync_copy(idx_hbm.at[pl.ds(base, b_per_w)], idx_v)

        def start_gather(c, buf, gsem):
            pltpu.make_async_copy(
                table_hbm.at[idx_v.at[pl.ds(c * _CHUNK, _CHUNK)]], buf, gsem
            ).start()

        for b in range(_NBUF):
            start_gather(b, rows[b], gsems[b])

        @pl.loop(0, n_chunks, step=_NBUF)
        def _(c):
            for b in range(_NBUF):
                cur = c + b
                buf, gsem, wsem = rows[b], gsems[b], wsems[b]
                pltpu.make_async_copy(
                    table_hbm.at[idx_v.at[pl.ds(0, _CHUNK)]], buf, gsem
                ).wait()
                pltpu.make_async_copy(
                    buf, out_hbm.at[pl.ds(base + cur * _CHUNK, _CHUNK)], wsem
                ).start()

                @pl.when(cur + _NBUF < n_chunks)
                def _():
                    pltpu.make_async_copy(
                        buf, out_hbm.at[pl.ds(base, _CHUNK)], wsem
                    ).wait()
                    start_gather(cur + _NBUF, buf, gsem)

        for b in range(_NBUF):
            pltpu.make_async_copy(
                rows[b], out_hbm.at[pl.ds(base, _CHUNK)], wsems[b]
            ).wait()

    return k(table, idx)


_BR = 1024


def _ln_body(seg_ref, emb_ref, pos_ref, segemb_ref, gamma_ref, beta_ref, out_ref):
    h = emb_ref[...]
    pos = pos_ref[...]
    posb = jnp.broadcast_to(pos[None], (_BR // _SEQ, _SEQ, _DIM)).reshape(_BR, _DIM)
    s = seg_ref[0, 0, :].astype(jnp.float32)
    se = segemb_ref[...]
    sadd = se[0][None, :] + s[:, None] * (se[1] - se[0])[None, :]
    h = h + posb + sadd
    mean = jnp.mean(h, axis=1, keepdims=True)
    hc = h - mean
    var = jnp.mean(hc * hc, axis=1, keepdims=True)
    out_ref[...] = hc * lax.rsqrt(var + 1e-5) * gamma_ref[...] + beta_ref[...]


def _ln_body_buf(buf_ref, seg_ref, emb_ref, pos_ref, segemb_ref, gamma_ref,
                 beta_ref, out_ref):
    del buf_ref
    _ln_body(seg_ref, emb_ref, pos_ref, segemb_ref, gamma_ref, beta_ref, out_ref)


_SIZES = [8192] + [16384] * 7 + [8192]


def _tc_ln_chunk(buf, emb_c, seg3, row_off, size, pos_embed, seg_embed,
                 gamma2, beta2):
    nblk_c = size // _BR
    blk0 = row_off // _BR
    common_in_specs = [
        pl.BlockSpec((1, 1, _BR), lambda i, b=blk0: (b + i, 0, 0)),
        pl.BlockSpec((_BR, _DIM), lambda i: (i, 0)),
        pl.BlockSpec((_SEQ, _DIM), lambda i: (0, 0)),
        pl.BlockSpec((2, _DIM), lambda i: (0, 0)),
        pl.BlockSpec((1, _DIM), lambda i: (0, 0)),
        pl.BlockSpec((1, _DIM), lambda i: (0, 0)),
    ]
    out_spec = pl.BlockSpec((_BR, _DIM), lambda i, b=blk0: (b + i, 0))
    out_shape = jax.ShapeDtypeStruct((_N, _DIM), jnp.float32)
    args = (seg3, emb_c, pos_embed, seg_embed, gamma2, beta2)
    cp = pltpu.CompilerParams(dimension_semantics=("parallel",))
    if buf is None:
        return pl.pallas_call(
            _ln_body,
            grid=(nblk_c,),
            in_specs=common_in_specs,
            out_specs=out_spec,
            out_shape=out_shape,
            compiler_params=cp,
        )(*args)
    return pl.pallas_call(
        _ln_body_buf,
        grid=(nblk_c,),
        in_specs=[pl.BlockSpec(memory_space=pl.ANY)] + common_in_specs,
        out_specs=out_spec,
        out_shape=out_shape,
        input_output_aliases={0: 0},
        compiler_params=cp,
    )(buf, *args)


@jax.jit
def kernel(x, seg, tok_embed, pos_embed, seg_embed, ln_gamma, ln_beta):
    xf = x.reshape(_N)
    seg3 = seg.reshape(_N // _BR, 1, _BR)
    gamma2 = ln_gamma.reshape(1, _DIM)
    beta2 = ln_beta.reshape(1, _DIM)
    offs = [0]
    for s in _SIZES:
        offs.append(offs[-1] + s)
    embs = [
        _sc_gather(tok_embed, lax.slice(xf, (off,), (off + size,)), size)
        for off, size in zip(offs, _SIZES)
    ]
    buf = None
    for emb_c, off, size in zip(embs, offs, _SIZES):
        buf = _tc_ln_chunk(buf, emb_c, seg3, off, size, pos_embed, seg_embed,
                           gamma2, beta2)
    return buf.reshape(_BATCH, _SEQ, _DIM)


# speedup vs baseline: 1.3524x; 1.0000x over previous
import functools

import jax
import jax.numpy as jnp
from jax import lax
from jax.experimental import pallas as pl
from jax.experimental.pallas import tpu as pltpu
from jax.experimental.pallas import tpu_sc as plsc

_VOCAB = 100000
_DIM = 768
_MAXLEN = 128
_BATCH = 1024
_SEQ = 128
_N = _BATCH * _SEQ

_NC = 2
_NS = 16
_NW = _NC * _NS
_B_PER_W = _N // _NW
_CHUNK = 32
_NBUF = 4


def _sc_gather(table, idx, n_rows):
    b_per_w = n_rows // _NW
    n_chunks = b_per_w // _CHUNK
    mesh = plsc.VectorSubcoreMesh(
        core_axis_name="c", subcore_axis_name="s", num_cores=_NC, num_subcores=_NS
    )

    @functools.partial(
        pl.kernel,
        out_type=jax.ShapeDtypeStruct((n_rows, _DIM), jnp.float32),
        mesh=mesh,
        scratch_types=[
            pltpu.VMEM((b_per_w,), jnp.int32),
        ]
        + [pltpu.VMEM((_CHUNK, _DIM), jnp.float32) for _ in range(_NBUF)]
        + [pltpu.SemaphoreType.DMA for _ in range(2 * _NBUF)],
    )
    def k(table_hbm, idx_hbm, out_hbm, idx_v, *bufs_and_sems):
        rows = bufs_and_sems[:_NBUF]
        gsems = bufs_and_sems[_NBUF : 2 * _NBUF]
        wsems = bufs_and_sems[2 * _NBUF :]
        wid = lax.axis_index("s") * _NC + lax.axis_index("c")
        base = wid * b_per_w
        pltpu.sync_copy(idx_hbm.at[pl.ds(base, b_per_w)], idx_v)

        def start_gather(c, buf, gsem):
            pltpu.make_async_copy(
                table_hbm.at[idx_v.at[pl.ds(c * _CHUNK, _CHUNK)]], buf, gsem
            ).start()

        for b in range(_NBUF):
            start_gather(b, rows[b], gsems[b])

        @pl.loop(0, n_chunks, step=_NBUF)
        def _(c):
            for b in range(_NBUF):
                cur = c + b
                buf, gsem, wsem = rows[b], gsems[b], wsems[b]
                pltpu.make_async_copy(
                    table_hbm.at[idx_v.at[pl.ds(0, _CHUNK)]], buf, gsem
                ).wait()
                pltpu.make_async_copy(
                    buf, out_hbm.at[pl.ds(base + cur * _CHUNK, _CHUNK)], wsem
                ).start()

                @pl.when(cur + _NBUF < n_chunks)
                def _():
                    pltpu.make_async_copy(
                        buf, out_hbm.at[pl.ds(base, _CHUNK)], wsem
                    ).wait()
                    start_gather(cur + _NBUF, buf, gsem)

        for b in range(_NBUF):
            pltpu.make_async_copy(
                rows[b], out_hbm.at[pl.ds(base, _CHUNK)], wsems[b]
            ).wait()

    return k(table, idx)


_BR = 2048


def _ln_body(seg_ref, emb_ref, pos_ref, segemb_ref, gamma_ref, beta_ref, out_ref):
    h = emb_ref[...]
    pos = pos_ref[...]
    posb = jnp.broadcast_to(pos[None], (_BR // _SEQ, _SEQ, _DIM)).reshape(_BR, _DIM)
    s = seg_ref[0, 0, :].astype(jnp.float32)
    se = segemb_ref[...]
    sadd = se[0][None, :] + s[:, None] * (se[1] - se[0])[None, :]
    h = h + posb + sadd
    mean = jnp.mean(h, axis=1, keepdims=True)
    hc = h - mean
    var = jnp.mean(hc * hc, axis=1, keepdims=True)
    out_ref[...] = hc * lax.rsqrt(var + 1e-5) * gamma_ref[...] + beta_ref[...]


def _ln_body_buf(buf_ref, seg_ref, emb_ref, pos_ref, segemb_ref, gamma_ref,
                 beta_ref, out_ref):
    del buf_ref
    _ln_body(seg_ref, emb_ref, pos_ref, segemb_ref, gamma_ref, beta_ref, out_ref)


_SIZES = [8192] + [16384] * 7 + [8192]


def _tc_ln_chunk(buf, emb_c, seg3, row_off, size, pos_embed, seg_embed,
                 gamma2, beta2):
    nblk_c = size // _BR
    blk0 = row_off // _BR
    common_in_specs = [
        pl.BlockSpec((1, 1, _BR), lambda i, b=blk0: (b + i, 0, 0)),
        pl.BlockSpec((_BR, _DIM), lambda i: (i, 0)),
        pl.BlockSpec((_SEQ, _DIM), lambda i: (0, 0)),
        pl.BlockSpec((2, _DIM), lambda i: (0, 0)),
        pl.BlockSpec((1, _DIM), lambda i: (0, 0)),
        pl.BlockSpec((1, _DIM), lambda i: (0, 0)),
    ]
    out_spec = pl.BlockSpec((_BR, _DIM), lambda i, b=blk0: (b + i, 0))
    out_shape = jax.ShapeDtypeStruct((_N, _DIM), jnp.float32)
    args = (seg3, emb_c, pos_embed, seg_embed, gamma2, beta2)
    cp = pltpu.CompilerParams(dimension_semantics=("parallel",))
    if buf is None:
        return pl.pallas_call(
            _ln_body,
            grid=(nblk_c,),
            in_specs=common_in_specs,
            out_specs=out_spec,
            out_shape=out_shape,
            compiler_params=cp,
        )(*args)
    return pl.pallas_call(
        _ln_body_buf,
        grid=(nblk_c,),
        in_specs=[pl.BlockSpec(memory_space=pl.ANY)] + common_in_specs,
        out_specs=out_spec,
        out_shape=out_shape,
        input_output_aliases={0: 0},
        compiler_params=cp,
    )(buf, *args)


@jax.jit
def kernel(x, seg, tok_embed, pos_embed, seg_embed, ln_gamma, ln_beta):
    xf = x.reshape(_N)
    seg3 = seg.reshape(_N // _BR, 1, _BR)
    gamma2 = ln_gamma.reshape(1, _DIM)
    beta2 = ln_beta.reshape(1, _DIM)
    offs = [0]
    for s in _SIZES:
        offs.append(offs[-1] + s)
    embs = [
        _sc_gather(tok_embed, lax.slice(xf, (off,), (off + size,)), size)
        for off, size in zip(offs, _SIZES)
    ]
    buf = None
    for emb_c, off, size in zip(embs, offs, _SIZES):
        buf = _tc_ln_chunk(buf, emb_c, seg3, off, size, pos_embed, seg_embed,
                           gamma2, beta2)
    return buf.reshape(_BATCH, _SEQ, _DIM)
